# baseline (speedup 1.0000x reference)
import functools

import jax
import jax.numpy as jnp
from jax import lax
from jax.experimental import pallas as pl
from jax.experimental.pallas import tpu as pltpu
from jax.experimental.pallas import tpu_sc as plsc

_NE = 64
_K = 8
_EPS = 1e-6
_NC = 2
_NS = 16
_NW = _NC * _NS
_L = 16


def _logits_kernel(x_ref, w_ref, b_ref, out_ref):
    out_ref[...] = lax.dot_general(
        x_ref[...], w_ref[...], (((1,), (1,)), ((), ())),
        preferred_element_type=jnp.float32) + b_ref[...]


@functools.partial(jax.jit, static_argnames=("block_t",))
def _logits_tc(x, w, b, block_t=1024):
    bsz, seq, d = x.shape
    n_tok = bsz * seq
    xf = x.reshape(n_tok, d)
    grid = n_tok // block_t
    return pl.pallas_call(
        _logits_kernel,
        grid=(grid,),
        in_specs=[
            pl.BlockSpec((block_t, d), lambda i: (i, 0)),
            pl.BlockSpec((_NE, d), lambda i: (0, 0)),
            pl.BlockSpec((1, _NE), lambda i: (0, 0)),
        ],
        out_specs=pl.BlockSpec((block_t, _NE), lambda i: (i, 0)),
        out_shape=jax.ShapeDtypeStruct((n_tok, _NE), jnp.float32),
    )(xf, w, b.reshape(1, _NE))


def _sc_gate(logits):
    n = logits.shape[0]
    bpw = n // _NW
    mesh = plsc.VectorSubcoreMesh(core_axis_name="c", subcore_axis_name="s",
                                  num_cores=_NC, num_subcores=_NS)

    @functools.partial(
        pl.kernel, mesh=mesh,
        out_type=jax.ShapeDtypeStruct((n, _NE), jnp.float32),
        compiler_params=pltpu.CompilerParams(needs_layout_passes=False),
        scratch_types=[
            pltpu.VMEM((bpw, _NE), jnp.float32),
            pltpu.VMEM((bpw, _NE), jnp.float32),
        ],
    )
    def gate(lg_hbm, out_hbm, lg_v, out_v):
        wid = lax.axis_index("s") * _NC + lax.axis_index("c")
        base = wid * bpw
        pltpu.sync_copy(lg_hbm.at[pl.ds(base, bpw)], lg_v)

        iota = lax.broadcasted_iota(jnp.int32, (_L,), 0)
        lane8 = iota < _K
        zeros16 = jnp.zeros((_L,), jnp.float32)
        perms = [iota ^ k for k in (1, 2, 4, 8)]

        def allreduce(v, op):
            for perm in perms:
                pv = v.at[perm].get(mode="promise_in_bounds")
                v = op(v, pv)
            return v

        def merge(a, ia, b, ib):
            rb = jnp.flip(b, 0)
            rib = jnp.flip(ib, 0)
            g = (a > rb) | ((a == rb) & (ia < rib))
            hk = jnp.where(g, a, rb)
            hv = jnp.where(g, ia, rib)
            return plsc.sort_key_val(hk, hv, descending=True)

        def body(t, carry):
            v0 = lg_v[t, pl.ds(0, _L)]
            v1 = lg_v[t, pl.ds(_L, _L)]
            v2 = lg_v[t, pl.ds(2 * _L, _L)]
            v3 = lg_v[t, pl.ds(3 * _L, _L)]
            m = allreduce(jnp.maximum(jnp.maximum(v0, v1),
                                      jnp.maximum(v2, v3)), jnp.maximum)
            ez = (jnp.exp(v0 - m) + jnp.exp(v1 - m)
                  + jnp.exp(v2 - m) + jnp.exp(v3 - m))
            z = allreduce(ez, jnp.add)
            s0k, s0i = plsc.sort_key_val(v0, iota, descending=True)
            s1k, s1i = plsc.sort_key_val(v1, iota + _L, descending=True)
            s2k, s2i = plsc.sort_key_val(v2, iota + 2 * _L, descending=True)
            s3k, s3i = plsc.sort_key_val(v3, iota + 3 * _L, descending=True)
            ak, ai = merge(s0k, s0i, s1k, s1i)
            bk, bi = merge(s2k, s2i, s3k, s3i)
            tk, tik = merge(ak, ai, bk, bi)
            te = jnp.exp(tk - m)
            s8 = allreduce(jnp.where(lane8, te, 0.0), jnp.add)
            scale = 1.0 / (s8 + _EPS * z)
            out_v[t, pl.ds(0, _L)] = zeros16
            out_v[t, pl.ds(_L, _L)] = zeros16
            out_v[t, pl.ds(2 * _L, _L)] = zeros16
            out_v[t, pl.ds(3 * _L, _L)] = zeros16
            tvec = jnp.full((_L,), t, jnp.int32)
            plsc.store_scatter(out_v, [tvec, tik], te * scale, mask=lane8)
            return carry

        lax.fori_loop(0, bpw, body, 0)
        pltpu.sync_copy(out_v, out_hbm.at[pl.ds(base, bpw)])

    return gate(logits)


@jax.jit
def _switch_gate_sc(x, w, b):
    bsz, seq, _ = x.shape
    logits = _logits_tc(x, w, b)
    return _sc_gate(logits).reshape(bsz, seq, _NE)


def kernel(x, W, b):
    return _switch_gate_sc(x, W, b)


# speedup vs baseline: 1.1894x; 1.1894x over previous
import functools

import jax
import jax.numpy as jnp
from jax import lax
from jax.experimental import pallas as pl
from jax.experimental.pallas import tpu as pltpu
from jax.experimental.pallas import tpu_sc as plsc

_NE = 64
_K = 8
_EPS = 1e-6
_NC = 2
_NS = 16
_NW = _NC * _NS
_L = 16


def _logits_kernel(x_ref, w_ref, b_ref, out_ref):
    out_ref[...] = lax.dot_general(
        x_ref[...], w_ref[...], (((1,), (1,)), ((), ())),
        preferred_element_type=jnp.float32) + b_ref[...]


@functools.partial(jax.jit, static_argnames=("block_t",))
def _logits_tc(x, w, b, block_t=1024):
    bsz, seq, d = x.shape
    n_tok = bsz * seq
    xf = x.reshape(n_tok, d)
    grid = n_tok // block_t
    return pl.pallas_call(
        _logits_kernel,
        grid=(grid,),
        in_specs=[
            pl.BlockSpec((block_t, d), lambda i: (i, 0)),
            pl.BlockSpec((_NE, d), lambda i: (0, 0)),
            pl.BlockSpec((1, _NE), lambda i: (0, 0)),
        ],
        out_specs=pl.BlockSpec((block_t, _NE), lambda i: (i, 0)),
        out_shape=jax.ShapeDtypeStruct((n_tok, _NE), jnp.float32),
    )(xf, w, b.reshape(1, _NE))


def _sc_gate(logits):
    n = logits.shape[0]
    bpw = n // _NW
    mesh = plsc.VectorSubcoreMesh(core_axis_name="c", subcore_axis_name="s",
                                  num_cores=_NC, num_subcores=_NS)

    @functools.partial(
        pl.kernel, mesh=mesh,
        out_type=jax.ShapeDtypeStruct((n, _NE), jnp.float32),
        compiler_params=pltpu.CompilerParams(needs_layout_passes=False),
        scratch_types=[
            pltpu.VMEM((bpw, _NE), jnp.float32),
        ],
    )
    def gate(lg_hbm, out_hbm, lg_v):
        out_v = lg_v
        wid = lax.axis_index("s") * _NC + lax.axis_index("c")
        base = wid * bpw
        pltpu.sync_copy(lg_hbm.at[pl.ds(base, bpw)], lg_v)

        iota = lax.broadcasted_iota(jnp.int32, (_L,), 0)
        lane8 = iota < _K
        zeros16 = jnp.zeros((_L,), jnp.float32)
        perms = [iota ^ k for k in (1, 2, 4, 8)]

        def allreduce(v, op):
            for perm in perms:
                pv = v.at[perm].get(mode="promise_in_bounds")
                v = op(v, pv)
            return v

        def merge(a, ia, b, ib):
            rb = jnp.flip(b, 0)
            rib = jnp.flip(ib, 0)
            g = (a > rb) | ((a == rb) & (ia < rib))
            hk = jnp.where(g, a, rb)
            hv = jnp.where(g, ia, rib)
            return plsc.sort_key_val(hk, hv, descending=True)

        lane0 = jnp.zeros((_L,), jnp.int32)

        @plsc.parallel_loop(0, bpw, unroll=4)
        def body(t):
            v0 = lg_v[t, pl.ds(0, _L)]
            v1 = lg_v[t, pl.ds(_L, _L)]
            v2 = lg_v[t, pl.ds(2 * _L, _L)]
            v3 = lg_v[t, pl.ds(3 * _L, _L)]
            s0k, s0i = plsc.sort_key_val(v0, iota, descending=True)
            s1k, s1i = plsc.sort_key_val(v1, iota + _L, descending=True)
            s2k, s2i = plsc.sort_key_val(v2, iota + 2 * _L, descending=True)
            s3k, s3i = plsc.sort_key_val(v3, iota + 3 * _L, descending=True)
            ak, ai = merge(s0k, s0i, s1k, s1i)
            bk, bi = merge(s2k, s2i, s3k, s3i)
            tk, tik = merge(ak, ai, bk, bi)
            m = tk.at[lane0].get(mode="promise_in_bounds")
            ez = (jnp.exp(v0 - m) + jnp.exp(v1 - m)
                  + jnp.exp(v2 - m) + jnp.exp(v3 - m))
            z = allreduce(ez, jnp.add)
            te = jnp.exp(tk - m)
            s8 = allreduce(jnp.where(lane8, te, 0.0), jnp.add)
            scale = 1.0 / (s8 + _EPS * z)
            out_v[t, pl.ds(0, _L)] = zeros16
            out_v[t, pl.ds(_L, _L)] = zeros16
            out_v[t, pl.ds(2 * _L, _L)] = zeros16
            out_v[t, pl.ds(3 * _L, _L)] = zeros16
            tvec = jnp.full((_L,), t, jnp.int32)
            plsc.store_scatter(out_v, [tvec, tik], te * scale, mask=lane8)
        pltpu.sync_copy(out_v, out_hbm.at[pl.ds(base, bpw)])

    return gate(logits)


@jax.jit
def _switch_gate_sc(x, w, b):
    bsz, seq, _ = x.shape
    logits = _logits_tc(x, w, b)
    return _sc_gate(logits).reshape(bsz, seq, _NE)


def kernel(x, W, b):
    return _switch_gate_sc(x, W, b)


# speedup vs baseline: 1.5844x; 1.3321x over previous
import functools

import jax
import jax.numpy as jnp
from jax import lax
from jax.experimental import pallas as pl

_NE = 64
_K = 8
_EPS = 1e-6


def _gate_kernel(x_ref, w_ref, b_ref, out_ref):
    x = x_ref[...]
    w = w_ref[...]
    logits = lax.dot_general(w, x, (((1,), (1,)), ((), ())),
                             preferred_element_type=jnp.float32)
    logits = logits + b_ref[...]
    t = logits.shape[1]
    idx = lax.broadcasted_iota(jnp.int32, (_NE, t), 0)
    m = jnp.max(logits, axis=0, keepdims=True)
    e = jnp.exp(logits - m)
    z = jnp.sum(e, axis=0, keepdims=True)
    tv = jnp.full((1, t), jnp.inf, jnp.float32)
    ti = jnp.full((1, t), -1, jnp.int32)
    for _ in range(_K):
        elig = (logits < tv) | ((logits == tv) & (idx > ti))
        lm = jnp.where(elig, logits, -jnp.inf)
        tv = jnp.max(lm, axis=0, keepdims=True)
        ti = jnp.min(jnp.where(lm == tv, idx, _NE), axis=0, keepdims=True)
    mask = (logits > tv) | ((logits == tv) & (idx <= ti))
    es = jnp.where(mask, e, 0.0)
    s8 = jnp.sum(es, axis=0, keepdims=True)
    out_ref[...] = es / (s8 + _EPS * z)


@functools.partial(jax.jit, static_argnames=("block_t",))
def _switch_gate(x, w, b, block_t=1024):
    bsz, seq, d = x.shape
    n_tok = bsz * seq
    xf = x.reshape(n_tok, d)
    grid = n_tok // block_t
    out_t = pl.pallas_call(
        _gate_kernel,
        grid=(grid,),
        in_specs=[
            pl.BlockSpec((block_t, d), lambda i: (i, 0)),
            pl.BlockSpec((_NE, d), lambda i: (0, 0)),
            pl.BlockSpec((_NE, 1), lambda i: (0, 0)),
        ],
        out_specs=pl.BlockSpec((_NE, block_t), lambda i: (0, i)),
        out_shape=jax.ShapeDtypeStruct((_NE, n_tok), jnp.float32),
    )(xf, w, b.reshape(_NE, 1))
    return out_t.T.reshape(bsz, seq, _NE)


def kernel(x, W, b):
    return _switch_gate(x, W, b, block_t=1024)
